# baseline (device time: 120336 ns/iter reference)
import jax
import jax.numpy as jnp
from jax import lax
from jax.experimental import pallas as pl
from jax.experimental.pallas import tpu as pltpu

K = 8


def kernel(partial, resid, gamma):
    m, d = resid.shape
    half = m // 2
    mb = half // K
    x2 = partial.reshape(m, d)
    gamma2 = gamma.reshape(1, d)

    def body(
        p_ref, r_hbm, g_ref, out_ref,
        r_buf, copy_sems, z_send, z_recv, y_send, y_recv,
    ):
        my_x = lax.axis_index("x")
        my_y = lax.axis_index("y")
        my_z = lax.axis_index("z")
        par = (my_x + my_y) % 2
        base = par * half
        zpeer = (my_x, my_y, 1 - my_z)
        ynbr = (my_x, 1 - my_y, my_z)

        barrier_sem = pltpu.get_barrier_semaphore()
        for nbr in (zpeer, ynbr):
            pl.semaphore_signal(
                barrier_sem, inc=1, device_id=nbr,
                device_id_type=pl.DeviceIdType.MESH,
            )
        pl.semaphore_wait(barrier_sem, 2)

        z_rdmas = []
        for k in range(K):
            rows = pl.ds(base + k * mb, mb)
            rdma = pltpu.make_async_remote_copy(
                src_ref=p_ref.at[rows, :],
                dst_ref=out_ref.at[rows, :],
                send_sem=z_send.at[k],
                recv_sem=z_recv.at[k],
                device_id=zpeer,
                device_id_type=pl.DeviceIdType.MESH,
            )
            rdma.start()
            z_rdmas.append(rdma)

        r_copy = pltpu.make_async_copy(
            r_hbm.at[pl.ds(base, half), :], r_buf, copy_sems.at[0]
        )
        r_copy.start()
        r_copy.wait()

        y_rdmas = []
        for k in range(K):
            rows = pl.ds(base + k * mb, mb)
            z_rdmas[k].wait_recv()
            y = p_ref[rows, :] + out_ref[rows, :] + r_buf[pl.ds(k * mb, mb), :]
            rms = jnp.sqrt(jnp.mean(y * y, axis=-1, keepdims=True) + 1e-6)
            out_ref[rows, :] = y / rms * g_ref[...]
            yr = pltpu.make_async_remote_copy(
                src_ref=out_ref.at[rows, :],
                dst_ref=out_ref.at[rows, :],
                send_sem=y_send.at[k],
                recv_sem=y_recv.at[k],
                device_id=ynbr,
                device_id_type=pl.DeviceIdType.MESH,
            )
            yr.start()
            y_rdmas.append(yr)

        for k in range(K):
            z_rdmas[k].wait_send()
            y_rdmas[k].wait_send()
            y_rdmas[k].wait_recv()

    return pl.pallas_call(
        body,
        out_shape=jax.ShapeDtypeStruct((m, d), jnp.float32),
        in_specs=[
            pl.BlockSpec(memory_space=pltpu.VMEM),
            pl.BlockSpec(memory_space=pl.ANY),
            pl.BlockSpec(memory_space=pltpu.VMEM),
        ],
        out_specs=pl.BlockSpec(memory_space=pltpu.VMEM),
        scratch_shapes=[
            pltpu.VMEM((half, d), jnp.float32),
            pltpu.SemaphoreType.DMA((2,)),
            pltpu.SemaphoreType.DMA((K,)),
            pltpu.SemaphoreType.DMA((K,)),
            pltpu.SemaphoreType.DMA((K,)),
            pltpu.SemaphoreType.DMA((K,)),
        ],
        compiler_params=pltpu.CompilerParams(collective_id=0),
    )(x2, resid, gamma2)


# device time: 112575 ns/iter; 1.0689x vs baseline; 1.0689x over previous
import jax
import jax.numpy as jnp
from jax import lax
from jax.experimental import pallas as pl
from jax.experimental.pallas import tpu as pltpu

K = 32


def kernel(partial, resid, gamma):
    m, d = resid.shape
    half = m // 2
    mb = half // K
    x2 = partial.reshape(m, d)
    gamma2 = gamma.reshape(1, d)

    def body(
        p_ref, r_hbm, g_ref, out_ref,
        r_buf, copy_sems, z_send, z_recv, y_send, y_recv,
    ):
        my_x = lax.axis_index("x")
        my_y = lax.axis_index("y")
        my_z = lax.axis_index("z")
        par = (my_x + my_y) % 2
        base = par * half
        zpeer = (my_x, my_y, 1 - my_z)
        ynbr = (my_x, 1 - my_y, my_z)

        barrier_sem = pltpu.get_barrier_semaphore()
        for nbr in (zpeer, ynbr):
            pl.semaphore_signal(
                barrier_sem, inc=1, device_id=nbr,
                device_id_type=pl.DeviceIdType.MESH,
            )
        pl.semaphore_wait(barrier_sem, 2)

        z_rdmas = []
        for k in range(K):
            rows = pl.ds(base + k * mb, mb)
            rdma = pltpu.make_async_remote_copy(
                src_ref=p_ref.at[rows, :],
                dst_ref=out_ref.at[rows, :],
                send_sem=z_send.at[k],
                recv_sem=z_recv.at[k],
                device_id=zpeer,
                device_id_type=pl.DeviceIdType.MESH,
            )
            rdma.start()
            z_rdmas.append(rdma)

        r_copy = pltpu.make_async_copy(
            r_hbm.at[pl.ds(base, half), :], r_buf, copy_sems.at[0]
        )
        r_copy.start()
        r_copy.wait()

        y_rdmas = []
        for k in range(K):
            rows = pl.ds(base + k * mb, mb)
            z_rdmas[k].wait_recv()
            y = p_ref[rows, :] + out_ref[rows, :] + r_buf[pl.ds(k * mb, mb), :]
            rms = jnp.sqrt(jnp.mean(y * y, axis=-1, keepdims=True) + 1e-6)
            out_ref[rows, :] = y / rms * g_ref[...]
            yr = pltpu.make_async_remote_copy(
                src_ref=out_ref.at[rows, :],
                dst_ref=out_ref.at[rows, :],
                send_sem=y_send.at[k],
                recv_sem=y_recv.at[k],
                device_id=ynbr,
                device_id_type=pl.DeviceIdType.MESH,
            )
            yr.start()
            y_rdmas.append(yr)

        for k in range(K):
            z_rdmas[k].wait_send()
            y_rdmas[k].wait_send()
            y_rdmas[k].wait_recv()

    return pl.pallas_call(
        body,
        out_shape=jax.ShapeDtypeStruct((m, d), jnp.float32),
        in_specs=[
            pl.BlockSpec(memory_space=pltpu.VMEM),
            pl.BlockSpec(memory_space=pl.ANY),
            pl.BlockSpec(memory_space=pltpu.VMEM),
        ],
        out_specs=pl.BlockSpec(memory_space=pltpu.VMEM),
        scratch_shapes=[
            pltpu.VMEM((half, d), jnp.float32),
            pltpu.SemaphoreType.DMA((2,)),
            pltpu.SemaphoreType.DMA((K,)),
            pltpu.SemaphoreType.DMA((K,)),
            pltpu.SemaphoreType.DMA((K,)),
            pltpu.SemaphoreType.DMA((K,)),
        ],
        compiler_params=pltpu.CompilerParams(collective_id=0),
    )(x2, resid, gamma2)
